# baseline (speedup 1.0000x reference)
import functools

import jax
import jax.numpy as jnp
from jax import lax
from jax.experimental import pallas as pl
from jax.experimental.pallas import tpu as pltpu
from jax.experimental.pallas import tpu_sc as plsc

_IDX_LANES = 128


def _make_gather(batch: int, seq: int, half: int):
    n_idx_rows = batch * seq // _IDX_LANES
    rows_per_batch = seq // _IDX_LANES
    info = plsc.get_sparse_core_info()
    num_workers = info.num_cores * info.num_subcores
    assert n_idx_rows % num_workers == 0, (n_idx_rows, num_workers)
    rows_per_worker = n_idx_rows // num_workers
    num_cores = info.num_cores

    out_sds = jax.ShapeDtypeStruct((batch, seq, half), jnp.float32)
    mesh = plsc.VectorSubcoreMesh(core_axis_name="c", subcore_axis_name="s")

    @functools.partial(
        pl.kernel,
        mesh=mesh,
        out_type=[out_sds, out_sds],
        scratch_types=[
            pltpu.VMEM((rows_per_worker, _IDX_LANES), jnp.int32),
            pltpu.VMEM((rows_per_worker, _IDX_LANES, 2 * half), jnp.float32),
            [pltpu.SemaphoreType.DMA] * rows_per_worker,
            pltpu.SemaphoreType.DMA,
        ],
        compiler_params=pltpu.CompilerParams(use_tc_tiling_on_sc=False),
    )
    def gather(tab_hbm, idx_hbm, cos_out, sin_out, idx_v, buf_v, gsems, ssem):
        wid = lax.axis_index("s") * num_cores + lax.axis_index("c")
        base = wid * rows_per_worker
        pltpu.sync_copy(idx_hbm.at[pl.ds(base, rows_per_worker)], idx_v)
        gathers = [
            pltpu.async_copy(tab_hbm.at[idx_v.at[j]], buf_v.at[j], gsems[j])
            for j in range(rows_per_worker)
        ]
        stores = []
        for j, g in enumerate(gathers):
            r = base + j
            b_idx = r // rows_per_batch
            s0 = (r % rows_per_batch) * _IDX_LANES
            g.wait()
            stores.append(pltpu.async_copy(
                buf_v.at[j, slice(None), pl.ds(0, half)],
                cos_out.at[b_idx, pl.ds(s0, _IDX_LANES)], ssem))
            stores.append(pltpu.async_copy(
                buf_v.at[j, slice(None), pl.ds(half, half)],
                sin_out.at[b_idx, pl.ds(s0, _IDX_LANES)], ssem))
        for st in stores:
            st.wait()

    return gather


def kernel(x, position_ids, cached_cos, cached_sin):
    del x
    b, s = position_ids.shape
    half = cached_cos.shape[3]
    assert (b * s) % _IDX_LANES == 0, (b, s)
    table = jnp.concatenate([cached_cos[0, 0], cached_sin[0, 0]], axis=-1)
    idx = position_ids.reshape(b * s // _IDX_LANES, _IDX_LANES).astype(jnp.int32)
    cos_r, sin_r = _make_gather(b, s, half)(table, idx)
    return (cos_r.reshape(b, s, 1, half), sin_r.reshape(b, s, 1, half))


# speedup vs baseline: 1.0266x; 1.0266x over previous
import functools

import jax
import jax.numpy as jnp
from jax import lax
from jax.experimental import pallas as pl
from jax.experimental.pallas import tpu as pltpu
from jax.experimental.pallas import tpu_sc as plsc

_IDX_LANES = 128


def _make_gather(batch: int, seq: int, half: int):
    n_idx_rows = batch * seq // _IDX_LANES
    info = plsc.get_sparse_core_info()
    num_workers = info.num_cores * info.num_subcores
    assert n_idx_rows % num_workers == 0, (n_idx_rows, num_workers)
    rows_per_worker = n_idx_rows // num_workers
    num_cores = info.num_cores

    out_sds = jax.ShapeDtypeStruct((batch * seq, 1, half), jnp.float32)
    mesh = plsc.VectorSubcoreMesh(core_axis_name="c", subcore_axis_name="s")

    @functools.partial(
        pl.kernel,
        mesh=mesh,
        out_type=[out_sds, out_sds],
        scratch_types=[
            pltpu.VMEM((rows_per_worker, _IDX_LANES), jnp.int32),
            pltpu.VMEM((rows_per_worker, _IDX_LANES, half), jnp.float32),
            pltpu.VMEM((rows_per_worker, _IDX_LANES, half), jnp.float32),
            [pltpu.SemaphoreType.DMA] * (2 * rows_per_worker),
            pltpu.SemaphoreType.DMA,
        ],
        compiler_params=pltpu.CompilerParams(use_tc_tiling_on_sc=False),
    )
    def gather(cos_hbm, sin_hbm, idx_hbm, cos_out, sin_out,
               idx_v, cos_v, sin_v, gsems, ssem):
        wid = lax.axis_index("s") * num_cores + lax.axis_index("c")
        base = wid * rows_per_worker
        pltpu.sync_copy(idx_hbm.at[pl.ds(base, rows_per_worker)], idx_v)
        gathers = []
        for j in range(rows_per_worker):
            gathers.append(
                (pltpu.async_copy(cos_hbm.at[idx_v.at[j]], cos_v.at[j], gsems[2 * j]),
                 pltpu.async_copy(sin_hbm.at[idx_v.at[j]], sin_v.at[j], gsems[2 * j + 1])))
        stores = []
        for j, (g_cos, g_sin) in enumerate(gathers):
            p0 = (base + j) * _IDX_LANES
            g_cos.wait()
            stores.append(pltpu.async_copy(
                cos_v.at[j], cos_out.at[pl.ds(p0, _IDX_LANES), 0], ssem))
            g_sin.wait()
            stores.append(pltpu.async_copy(
                sin_v.at[j], sin_out.at[pl.ds(p0, _IDX_LANES), 0], ssem))
        for st in stores:
            st.wait()

    return gather


def kernel(x, position_ids, cached_cos, cached_sin):
    del x
    b, s = position_ids.shape
    max_pos, half = cached_cos.shape[2], cached_cos.shape[3]
    assert (b * s) % _IDX_LANES == 0, (b, s)
    idx = position_ids.reshape(b * s // _IDX_LANES, _IDX_LANES).astype(jnp.int32)
    cos_tab = cached_cos.reshape(max_pos, half)
    sin_tab = cached_sin.reshape(max_pos, half)
    cos_r, sin_r = _make_gather(b, s, half)(cos_tab, sin_tab, idx)
    return (cos_r.reshape(b, s, 1, half), sin_r.reshape(b, s, 1, half))
